# baseline (speedup 1.0000x reference)
import functools

import jax
import jax.numpy as jnp
from jax import lax
from jax.experimental import pallas as pl
from jax.experimental.pallas import tpu as pltpu
from jax.experimental.pallas import tpu_sc as plsc

_N_NODES = 10000
_N_EDGES = 320000
_D = 128
_ROWS = 1024
_NC = 2
_NS = 16
_NW = _NC * _NS
_EPT = _N_EDGES // _NW
_CH = _EPT // 16
_TRASH = _ROWS
_SROWS = 1152
_RPT = _SROWS // _NS
_WIDTH = 144


def _logits_body(asrc_ref, adst_ref, w_ref, x_ref, out_ref, xbf_ref):
    att2 = jnp.concatenate([asrc_ref[...], adst_ref[...]], axis=0)
    wa = lax.dot_general(att2, w_ref[...], (((1,), (1,)), ((), ())),
                         preferred_element_type=jnp.float32)
    out_ref[...] = lax.dot_general(wa, x_ref[...], (((1,), (1,)), ((), ())),
                                   preferred_element_type=jnp.float32)
    xbf_ref[...] = x_ref[...].astype(jnp.bfloat16)


def _node_logits(att_src, att_dst, W, x):
    return pl.pallas_call(
        _logits_body,
        in_specs=[
            pl.BlockSpec((1, _D), lambda: (0, 0)),
            pl.BlockSpec((1, _D), lambda: (0, 0)),
            pl.BlockSpec((_D, _D), lambda: (0, 0)),
            pl.BlockSpec((_N_NODES, _D), lambda: (0, 0)),
        ],
        out_specs=[
            pl.BlockSpec((2, _N_NODES), lambda: (0, 0)),
            pl.BlockSpec((_N_NODES, _D), lambda: (0, 0)),
        ],
        out_shape=[
            jax.ShapeDtypeStruct((2, _N_NODES), jnp.float32),
            jax.ShapeDtypeStruct((_N_NODES, _D), jnp.bfloat16),
        ],
    )(att_src, att_dst, W, x)


def _sc_body(ei_hbm, a8_hbm, x_hbm, feat_hbm, den_hbm,
             srcv, dstv, asrcv, adstv, comp_s, comp_d,
             rows_v, scaled_v, agg_sh,
             gsem0, gsem1, gsem2, gsem3, gsem4, gsem5, gsem6, gsem7,
             ssem0, ssem1):
    c = lax.axis_index("c")
    s = lax.axis_index("s")
    wid = c * _NS + s
    base = wid * _EPT
    zf16 = jnp.zeros((16,), jnp.float32)
    gsems = (gsem0, gsem1, gsem2, gsem3, gsem4, gsem5, gsem6, gsem7)
    ssems = (ssem0, ssem1)

    with jax.named_scope("sc_stage"):
        cp0 = pltpu.make_async_copy(ei_hbm.at[0, pl.ds(base, _EPT)], srcv, gsem0)
        cp1 = pltpu.make_async_copy(ei_hbm.at[1, pl.ds(base, _EPT)], dstv, gsem1)
        cp2 = pltpu.make_async_copy(a8_hbm.at[0], asrcv, gsem2)
        cp3 = pltpu.make_async_copy(a8_hbm.at[1, pl.ds(0, _ROWS)],
                                    adstv.at[pl.ds(0, _ROWS)], gsem3)
        cp0.start(); cp1.start(); cp2.start(); cp3.start()
        adstv[pl.ds(_ROWS, 16)] = zf16

        for l in range(16):
            for cc in range(_WIDTH // 16):
                scaled_v[0, l, pl.ds(cc * 16, 16)] = zf16
        for k in range(4):
            pltpu.sync_copy(scaled_v.at[0, pl.ds(0, 16)],
                            agg_sh.at[pl.ds(s * _RPT + k * 16, 16)])
        pltpu.sync_copy(scaled_v.at[0, pl.ds(0, 8)],
                        agg_sh.at[pl.ds(s * _RPT + 64, 8)])
        cp0.wait(); cp1.wait(); cp2.wait(); cp3.wait()
        plsc.subcore_barrier()

        _U1 = 5
        assert _CH % _U1 == 0

        def p1(i2, off):
            for u in range(_U1):
                i = i2 * _U1 + u
                d16 = dstv[pl.ds(i * 16, 16)]
                s16 = srcv[pl.ds(i * 16, 16)]
                valid = d16 < _ROWS
                ds_, ss_ = plsc.sort_key_val(d16, s16)
                comp_s[pl.ds(off, 16)] = ss_
                comp_d[pl.ds(off, 16)] = ds_
                cntv = plsc.all_reduce_population_count(valid)
                off = off + cntv[0]
            return off

        with jax.named_scope("sc_pass1"):
            n = lax.fori_loop(0, _CH // _U1, p1, jnp.int32(0))
            comp_s[pl.ds(n, 16)] = jnp.zeros((16,), jnp.int32)
            comp_d[pl.ds(n, 16)] = jnp.full((16,), _TRASH, jnp.int32)
        nch = (n + 15) // 16

        def g_issue(k, r):
            idx = comp_s[pl.ds(k * 16, 16)]
            pltpu.make_async_copy(x_hbm.at[idx], rows_v.at[r], gsems[r]).start()

        def g_wait(k, r):
            idx = comp_s[pl.ds(k * 16, 16)]
            pltpu.make_async_copy(x_hbm.at[idx], rows_v.at[r], gsems[r]).wait()

        lane0 = lax.iota(jnp.int32, 16) == 0
        dnums = lax.GatherDimensionNumbers(
            offset_dims=(), collapsed_slice_dims=(0,), start_index_map=(0,))

        def s_desc(k, p):
            idx = comp_d[pl.ds(k * 16, 16)]
            return pltpu.make_async_copy(scaled_v.at[p], agg_sh.at[idx],
                                         ssems[p])

        def process(k, r, p):
            s16 = comp_s[pl.ds(k * 16, 16)]
            d16 = comp_d[pl.ds(k * 16, 16)]
            a_s = plsc.load_gather(asrcv, [s16])
            a_d = plsc.load_gather(adstv, [d16])
            e = a_s + a_d
            e = jnp.where(e >= 0.0, e, e * jnp.float32(0.2))
            wv = jnp.exp(e)
            g_wait(k, r)

            @pl.when(k >= 2)
            def _():
                s_desc(k - 2, p).wait()

            for l in range(16):
                wl = lax.gather(wv, jnp.full((16, 1), l, jnp.int32), dnums,
                                slice_sizes=(1,),
                                mode=lax.GatherScatterMode.PROMISE_IN_BOUNDS)
                for cc in range(_D // 32):
                    v32 = rows_v[r, l, pl.ds(cc * 32, 32)]
                    va, vb = plsc.unpack(v32,
                                         format=plsc.PackFormat.INTERLEAVED)
                    scaled_v[p, l, pl.ds(cc * 32, 16)] = va * wl
                    scaled_v[p, l, pl.ds(cc * 32 + 16, 16)] = vb * wl
                scaled_v[p, l, pl.ds(_D, 16)] = jnp.where(lane0, wl, 0.0)
            pltpu.async_copy(scaled_v.at[p], agg_sh.at[d16], ssems[p],
                             add=True)

        _DEPTH = 8
        with jax.named_scope("sc_pass2"):
            for r in range(_DEPTH):

                @pl.when(r < nch)
                def _():
                    g_issue(jnp.int32(r), r)

            def p2(k4, carry):
                for r in range(_DEPTH):
                    k = k4 * _DEPTH + r
                    p = r % 2

                    @pl.when(k < nch)
                    def _():
                        process(k, r, p)

                        @pl.when(k + _DEPTH < nch)
                        def _():
                            g_issue(k + _DEPTH, r)

                return carry

            lax.fori_loop(0, (nch + _DEPTH - 1) // _DEPTH, p2, jnp.int32(0))

            @pl.when(nch >= 2)
            def _():
                s_desc(jnp.int32(0), 0).wait()
                s_desc(jnp.int32(0), 1).wait()

            @pl.when(nch == 1)
            def _():
                s_desc(jnp.int32(0), 0).wait()

        plsc.subcore_barrier()

        pltpu.sync_copy(agg_sh.at[pl.ds(s * _RPT, _RPT), pl.ds(0, _D)],
                        feat_hbm.at[c, pl.ds(s * _RPT, _RPT)])
        pltpu.sync_copy(agg_sh.at[pl.ds(s * _RPT, _RPT), pl.ds(_D, 16)],
                        den_hbm.at[c, pl.ds(s * _RPT, _RPT)])


def _sc_edge_agg(ei, a8, x):
    mesh = plsc.VectorSubcoreMesh(core_axis_name="c", subcore_axis_name="s")
    f = pl.kernel(
        _sc_body,
        mesh=mesh,
        out_type=(jax.ShapeDtypeStruct((_NC, _SROWS, _D), jnp.float32),
                  jax.ShapeDtypeStruct((_NC, _SROWS, 16), jnp.float32)),
        scratch_types=[
            pltpu.VMEM((_EPT,), jnp.int32),
            pltpu.VMEM((_EPT,), jnp.int32),
            pltpu.VMEM((_N_NODES,), jnp.float32),
            pltpu.VMEM((_ROWS + 16,), jnp.float32),
            pltpu.VMEM((_EPT + 16,), jnp.int32),
            pltpu.VMEM((_EPT + 16,), jnp.int32),
            pltpu.VMEM((8, 16, _D), jnp.bfloat16),
            pltpu.VMEM((2, 16, _WIDTH), jnp.float32),
            pltpu.VMEM_SHARED((_SROWS, _WIDTH), jnp.float32),
        ] + [pltpu.SemaphoreType.DMA] * 10,
        compiler_params=pltpu.CompilerParams(needs_layout_passes=False,
                                             use_tc_tiling_on_sc=False),
    )
    return f(ei, a8, x)


def _finish_body(feat_any, den_any, w_ref, bias_ref, mask_ref, y_ref,
                 feat_v, den_v, fsem, dsem):
    pltpu.make_async_copy(feat_any, feat_v, fsem).start()
    pltpu.make_async_copy(den_any, den_v, dsem).start()
    pltpu.make_async_copy(feat_any, feat_v, fsem).wait()
    pltpu.make_async_copy(den_any, den_v, dsem).wait()
    numer = feat_v[0, 0:_ROWS, :] + feat_v[1, 0:_ROWS, :]
    den2 = den_v[0, 0:_ROWS, :] + den_v[1, 0:_ROWS, :]
    denom = jnp.sum(den2, axis=1, keepdims=True)
    rows = lax.dot_general(numer, w_ref[...], (((1,), (0,)), ((), ())),
                           preferred_element_type=jnp.float32)
    out = rows / (denom + jnp.float32(1e-16)) + bias_ref[...]
    xo = jnp.where(out > 0, out, jnp.exp(out) - 1.0)
    x0 = xo[0:1, :]
    sdot = lax.dot_general(x0, xo, (((1,), (1,)), ((), ())),
                           preferred_element_type=jnp.float32)
    sm = sdot * mask_ref[...]
    y = jnp.where(sm > 0, sm, jnp.exp(sm) - 1.0)
    y_ref[...] = y[:, 1:_ROWS]


def _finish(featv, denv, W, bias, mask_row):
    return pl.pallas_call(
        _finish_body,
        in_specs=[
            pl.BlockSpec(memory_space=pl.ANY),
            pl.BlockSpec(memory_space=pl.ANY),
            pl.BlockSpec((_D, _D), lambda: (0, 0)),
            pl.BlockSpec((1, _D), lambda: (0, 0)),
            pl.BlockSpec((1, _ROWS), lambda: (0, 0)),
        ],
        scratch_shapes=[
            pltpu.VMEM((_NC, _SROWS, _D), jnp.float32),
            pltpu.VMEM((_NC, _SROWS, 16), jnp.float32),
            pltpu.SemaphoreType.DMA,
            pltpu.SemaphoreType.DMA,
        ],
        out_specs=pl.BlockSpec((1, _ROWS - 1), lambda: (0, 0)),
        out_shape=jax.ShapeDtypeStruct((1, _ROWS - 1), jnp.float32),
    )(featv, denv, W, bias, mask_row)


def kernel(x, edge_index, item_len, W, att_src, att_dst, bias):
    a8, xbf = _node_logits(att_src[None, :], att_dst[None, :], W, x)
    feat, den = _sc_edge_agg(edge_index.astype(jnp.int32), a8, xbf)
    perm = []
    for cc in range(_D // 32):
        perm += [cc * 32 + 2 * i for i in range(16)]
        perm += [cc * 32 + 2 * i + 1 for i in range(16)]
    Wp = W[jnp.array(perm, dtype=jnp.int32), :]
    mask_row = (jnp.arange(_ROWS, dtype=jnp.int32) < item_len).astype(
        jnp.float32)[None, :]
    y2 = _finish(feat, den, Wp, bias[None, :], mask_row)
    return (x, y2[0])


# speedup vs baseline: 1.0708x; 1.0708x over previous
import functools

import jax
import jax.numpy as jnp
from jax import lax
from jax.experimental import pallas as pl
from jax.experimental.pallas import tpu as pltpu
from jax.experimental.pallas import tpu_sc as plsc

_N_NODES = 10000
_N_EDGES = 320000
_D = 128
_ROWS = 1024
_NC = 2
_NS = 16
_NW = _NC * _NS
_EPT = _N_EDGES // _NW
_CH = _EPT // 16
_TRASH = _ROWS
_SROWS = 1152
_RPT = _SROWS // _NS
_WIDTH = 144


def _logits_body(asrc_ref, adst_ref, w_ref, x_ref, out_ref):
    att2 = jnp.concatenate([asrc_ref[...], adst_ref[...]], axis=0)
    wa = lax.dot_general(att2, w_ref[...], (((1,), (1,)), ((), ())),
                         preferred_element_type=jnp.float32)
    out_ref[...] = lax.dot_general(wa, x_ref[...], (((1,), (1,)), ((), ())),
                                   preferred_element_type=jnp.float32)


def _node_logits(att_src, att_dst, W, x):
    return pl.pallas_call(
        _logits_body,
        in_specs=[
            pl.BlockSpec((1, _D), lambda: (0, 0)),
            pl.BlockSpec((1, _D), lambda: (0, 0)),
            pl.BlockSpec((_D, _D), lambda: (0, 0)),
            pl.BlockSpec((_N_NODES, _D), lambda: (0, 0)),
        ],
        out_specs=pl.BlockSpec((2, _N_NODES), lambda: (0, 0)),
        out_shape=jax.ShapeDtypeStruct((2, _N_NODES), jnp.float32),
    )(att_src, att_dst, W, x)


def _sc_body(ei_hbm, a8_hbm, x_hbm, feat_hbm, den_hbm,
             srcv, dstv, asrcv, adstv, comp_s, comp_d,
             rows_v, scaled_v, agg_sh,
             gsem0, gsem1, gsem2, gsem3, gsem4, gsem5, gsem6, gsem7,
             ssem0, ssem1):
    c = lax.axis_index("c")
    s = lax.axis_index("s")
    wid = c * _NS + s
    base = wid * _EPT
    zf16 = jnp.zeros((16,), jnp.float32)
    gsems = (gsem0, gsem1, gsem2, gsem3, gsem4, gsem5, gsem6, gsem7)
    ssems = (ssem0, ssem1)

    with jax.named_scope("sc_stage"):
        cp0 = pltpu.make_async_copy(ei_hbm.at[0, pl.ds(base, _EPT)], srcv, gsem0)
        cp1 = pltpu.make_async_copy(ei_hbm.at[1, pl.ds(base, _EPT)], dstv, gsem1)
        cp2 = pltpu.make_async_copy(a8_hbm.at[0], asrcv, gsem2)
        cp3 = pltpu.make_async_copy(a8_hbm.at[1, pl.ds(0, _ROWS)],
                                    adstv.at[pl.ds(0, _ROWS)], gsem3)
        cp0.start(); cp1.start(); cp2.start(); cp3.start()
        adstv[pl.ds(_ROWS, 16)] = zf16

        for l in range(16):
            for cc in range(_WIDTH // 16):
                scaled_v[0, l, pl.ds(cc * 16, 16)] = zf16
        for k in range(4):
            pltpu.sync_copy(scaled_v.at[0, pl.ds(0, 16)],
                            agg_sh.at[pl.ds(s * _RPT + k * 16, 16)])
        pltpu.sync_copy(scaled_v.at[0, pl.ds(0, 8)],
                        agg_sh.at[pl.ds(s * _RPT + 64, 8)])
        cp0.wait(); cp1.wait(); cp2.wait(); cp3.wait()
        plsc.subcore_barrier()

        _U1 = 5
        assert _CH % _U1 == 0

        def p1(i2, off):
            for u in range(_U1):
                i = i2 * _U1 + u
                d16 = dstv[pl.ds(i * 16, 16)]
                s16 = srcv[pl.ds(i * 16, 16)]
                valid = d16 < _ROWS
                ds_, ss_ = plsc.sort_key_val(d16, s16)
                comp_s[pl.ds(off, 16)] = ss_
                comp_d[pl.ds(off, 16)] = ds_
                cntv = plsc.all_reduce_population_count(valid)
                off = off + cntv[0]
            return off

        with jax.named_scope("sc_pass1"):
            n = lax.fori_loop(0, _CH // _U1, p1, jnp.int32(0))
            comp_s[pl.ds(n, 16)] = jnp.zeros((16,), jnp.int32)
            comp_d[pl.ds(n, 16)] = jnp.full((16,), _TRASH, jnp.int32)
        nch = (n + 15) // 16

        def g_issue(k, r):
            idx = comp_s[pl.ds(k * 16, 16)]
            pltpu.make_async_copy(x_hbm.at[idx], rows_v.at[r], gsems[r]).start()

        def g_wait(k, r):
            idx = comp_s[pl.ds(k * 16, 16)]
            pltpu.make_async_copy(x_hbm.at[idx], rows_v.at[r], gsems[r]).wait()

        lane0 = lax.iota(jnp.int32, 16) == 0
        dnums = lax.GatherDimensionNumbers(
            offset_dims=(), collapsed_slice_dims=(0,), start_index_map=(0,))

        def s_desc(k, p):
            idx = comp_d[pl.ds(k * 16, 16)]
            return pltpu.make_async_copy(scaled_v.at[p], agg_sh.at[idx],
                                         ssems[p])

        def process(k, r, p):
            s16 = comp_s[pl.ds(k * 16, 16)]
            d16 = comp_d[pl.ds(k * 16, 16)]
            a_s = plsc.load_gather(asrcv, [s16])
            a_d = plsc.load_gather(adstv, [d16])
            e = a_s + a_d
            e = jnp.where(e >= 0.0, e, e * jnp.float32(0.2))
            wv = jnp.exp(e)
            g_wait(k, r)

            @pl.when(k >= 2)
            def _():
                s_desc(k - 2, p).wait()

            for l in range(16):
                wl = lax.gather(wv, jnp.full((16, 1), l, jnp.int32), dnums,
                                slice_sizes=(1,),
                                mode=lax.GatherScatterMode.PROMISE_IN_BOUNDS)
                for cc in range(_D // 16):
                    scaled_v[p, l, pl.ds(cc * 16, 16)] = (
                        rows_v[r, l, pl.ds(cc * 16, 16)] * wl)
                scaled_v[p, l, pl.ds(_D, 16)] = jnp.where(lane0, wl, 0.0)
            pltpu.async_copy(scaled_v.at[p], agg_sh.at[d16], ssems[p],
                             add=True)

        _DEPTH = 8
        with jax.named_scope("sc_pass2"):
            for r in range(_DEPTH):

                @pl.when(r < nch)
                def _():
                    g_issue(jnp.int32(r), r)

            def p2(k4, carry):
                for r in range(_DEPTH):
                    k = k4 * _DEPTH + r
                    p = r % 2

                    @pl.when(k < nch)
                    def _():
                        process(k, r, p)

                        @pl.when(k + _DEPTH < nch)
                        def _():
                            g_issue(k + _DEPTH, r)

                return carry

            lax.fori_loop(0, (nch + _DEPTH - 1) // _DEPTH, p2, jnp.int32(0))

            @pl.when(nch >= 2)
            def _():
                s_desc(jnp.int32(0), 0).wait()
                s_desc(jnp.int32(0), 1).wait()

            @pl.when(nch == 1)
            def _():
                s_desc(jnp.int32(0), 0).wait()

        plsc.subcore_barrier()

        pltpu.sync_copy(agg_sh.at[pl.ds(s * _RPT, _RPT), pl.ds(0, _D)],
                        feat_hbm.at[c, pl.ds(s * _RPT, _RPT)])
        pltpu.sync_copy(agg_sh.at[pl.ds(s * _RPT, _RPT), pl.ds(_D, 16)],
                        den_hbm.at[c, pl.ds(s * _RPT, _RPT)])


def _sc_edge_agg(ei, a8, x):
    mesh = plsc.VectorSubcoreMesh(core_axis_name="c", subcore_axis_name="s")
    f = pl.kernel(
        _sc_body,
        mesh=mesh,
        out_type=(jax.ShapeDtypeStruct((_NC, _SROWS, _D), jnp.float32),
                  jax.ShapeDtypeStruct((_NC, _SROWS, 16), jnp.float32)),
        scratch_types=[
            pltpu.VMEM((_EPT,), jnp.int32),
            pltpu.VMEM((_EPT,), jnp.int32),
            pltpu.VMEM((_N_NODES,), jnp.float32),
            pltpu.VMEM((_ROWS + 16,), jnp.float32),
            pltpu.VMEM((_EPT + 16,), jnp.int32),
            pltpu.VMEM((_EPT + 16,), jnp.int32),
            pltpu.VMEM((8, 16, _D), jnp.float32),
            pltpu.VMEM((2, 16, _WIDTH), jnp.float32),
            pltpu.VMEM_SHARED((_SROWS, _WIDTH), jnp.float32),
        ] + [pltpu.SemaphoreType.DMA] * 10,
        compiler_params=pltpu.CompilerParams(needs_layout_passes=False,
                                             use_tc_tiling_on_sc=False),
    )
    return f(ei, a8, x)


def _finish_body(feat_any, den_any, w_ref, bias_ref, mask_ref, y_ref,
                 feat_v, den_v, fsem, dsem):
    pltpu.make_async_copy(feat_any, feat_v, fsem).start()
    pltpu.make_async_copy(den_any, den_v, dsem).start()
    pltpu.make_async_copy(feat_any, feat_v, fsem).wait()
    pltpu.make_async_copy(den_any, den_v, dsem).wait()
    numer = feat_v[0, 0:_ROWS, :] + feat_v[1, 0:_ROWS, :]
    den2 = den_v[0, 0:_ROWS, :] + den_v[1, 0:_ROWS, :]
    denom = jnp.sum(den2, axis=1, keepdims=True)
    rows = lax.dot_general(numer, w_ref[...], (((1,), (0,)), ((), ())),
                           preferred_element_type=jnp.float32)
    out = rows / (denom + jnp.float32(1e-16)) + bias_ref[...]
    xo = jnp.where(out > 0, out, jnp.exp(out) - 1.0)
    x0 = xo[0:1, :]
    sdot = lax.dot_general(x0, xo, (((1,), (1,)), ((), ())),
                           preferred_element_type=jnp.float32)
    sm = sdot * mask_ref[...]
    y = jnp.where(sm > 0, sm, jnp.exp(sm) - 1.0)
    y_ref[...] = y[:, 1:_ROWS]


def _finish(featv, denv, W, bias, mask_row):
    return pl.pallas_call(
        _finish_body,
        in_specs=[
            pl.BlockSpec(memory_space=pl.ANY),
            pl.BlockSpec(memory_space=pl.ANY),
            pl.BlockSpec((_D, _D), lambda: (0, 0)),
            pl.BlockSpec((1, _D), lambda: (0, 0)),
            pl.BlockSpec((1, _ROWS), lambda: (0, 0)),
        ],
        scratch_shapes=[
            pltpu.VMEM((_NC, _SROWS, _D), jnp.float32),
            pltpu.VMEM((_NC, _SROWS, 16), jnp.float32),
            pltpu.SemaphoreType.DMA,
            pltpu.SemaphoreType.DMA,
        ],
        out_specs=pl.BlockSpec((1, _ROWS - 1), lambda: (0, 0)),
        out_shape=jax.ShapeDtypeStruct((1, _ROWS - 1), jnp.float32),
    )(featv, denv, W, bias, mask_row)


def kernel(x, edge_index, item_len, W, att_src, att_dst, bias):
    a8 = _node_logits(att_src[None, :], att_dst[None, :], W, x)
    feat, den = _sc_edge_agg(edge_index.astype(jnp.int32), a8, x)
    mask_row = (jnp.arange(_ROWS, dtype=jnp.int32) < item_len).astype(
        jnp.float32)[None, :]
    y2 = _finish(feat, den, W, bias[None, :], mask_row)
    return (x, y2[0])
